# baseline (speedup 1.0000x reference)
import functools
import jax
import jax.numpy as jnp
from jax import lax
from jax.experimental import pallas as pl
from jax.experimental.pallas import tpu as pltpu
from jax.experimental.pallas import tpu_sc as plsc

TOKENS = 32768
SEGS = 16
NODE_D = 512
GOAL_D = 128
CH = 1024
NBUF = 8
NCH = TOKENS // CH
OUTER = NCH // NBUF
WEIGHT = 0.7

WIN = 2064
MAX_BASE = TOKENS - WIN
LANES = 16


def _tc_body(w1n_ref, w1g_ref, b1_ref, w2_ref, b2_ref,
             nodes_hbm, goal_hbm, out_ref,
             nbuf_ref, gbuf_ref, scores_ref, sems):

    def node_copy(c, b):
        return pltpu.make_async_copy(
            nodes_hbm.at[pl.ds(c * CH, CH), :], nbuf_ref.at[b], sems.at[b, 0])

    def goal_copy(c, b):
        return pltpu.make_async_copy(
            goal_hbm.at[pl.ds(c * CH, CH), :], gbuf_ref.at[b], sems.at[b, 1])

    for b in range(NBUF):
        node_copy(b, b).start()
        goal_copy(b, b).start()

    def outer(i, _):
        for b in range(NBUF):
            c = i * NBUF + b
            node_copy(c, b).wait()
            goal_copy(c, b).wait()

            h = jnp.dot(nbuf_ref[b], w1n_ref[...],
                        preferred_element_type=jnp.float32)
            h = h + jnp.dot(gbuf_ref[b], w1g_ref[...],
                            preferred_element_type=jnp.float32)
            h = jnp.maximum(h + b1_ref[...], 0.0)
            scores = (jnp.sum(h * w2_ref[...], axis=1, keepdims=True)
                      + b2_ref[...])
            scores_ref[pl.ds(c * (CH // 128), CH // 128), :] = (
                scores.reshape(CH // 128, 128))

            @pl.when(i < OUTER - 1)
            def _prefetch():
                node_copy(c + NBUF, b).start()
                goal_copy(c + NBUF, b).start()
        return 0

    lax.fori_loop(0, OUTER, outer, 0)
    out_ref[...] = scores_ref[...]


def _tc_scores(nodes, goal, W1, b1, W2, b2):
    w1nT = W1[:, :NODE_D].T
    w1gT = W1[:, NODE_D:].T

    full = lambda i: (0, 0)
    out = pl.pallas_call(
        _tc_body,
        grid=(1,),
        in_specs=[
            pl.BlockSpec((NODE_D, SEGS), full),
            pl.BlockSpec((GOAL_D, SEGS), full),
            pl.BlockSpec((1, SEGS), full),
            pl.BlockSpec((1, SEGS), full),
            pl.BlockSpec((1, 1), full),
            pl.BlockSpec(memory_space=pl.ANY),
            pl.BlockSpec(memory_space=pl.ANY),
        ],
        out_specs=pl.BlockSpec((TOKENS // 128, 128), full),
        out_shape=jax.ShapeDtypeStruct((TOKENS // 128, 128), jnp.float32),
        scratch_shapes=[
            pltpu.VMEM((NBUF, CH, NODE_D), jnp.float32),
            pltpu.VMEM((NBUF, CH, GOAL_D), jnp.float32),
            pltpu.VMEM((TOKENS // 128, 128), jnp.float32),
            pltpu.SemaphoreType.DMA((NBUF, 2)),
        ],
        compiler_params=pltpu.CompilerParams(
            dimension_semantics=("arbitrary",)),
    )(w1nT, w1gT, b1.reshape(1, SEGS), W2.reshape(1, SEGS),
      b2.reshape(1, 1), nodes, goal)
    return out.reshape(TOKENS)


def _sc_pool(scores, starts, ends):
    mesh = plsc.VectorSubcoreMesh(core_axis_name="c", subcore_axis_name="s")

    @functools.partial(
        pl.kernel, mesh=mesh,
        out_type=jax.ShapeDtypeStruct((SEGS, LANES), jnp.float32),
        scratch_types=[
            pltpu.VMEM((WIN,), jnp.float32),
            pltpu.VMEM((LANES,), jnp.int32),
            pltpu.VMEM((LANES,), jnp.int32),
            pltpu.VMEM((LANES,), jnp.float32),
            pltpu.VMEM((LANES,), jnp.float32),
            pltpu.VMEM((LANES,), jnp.float32),
            pltpu.SMEM((2,), jnp.int32),
        ],
    )
    def k(scores_hbm, starts_hbm, ends_hbm, out_hbm, win_v, sv, ev, res_v,
          s_ref, m_ref, se_smem):
        cid = lax.axis_index("c")
        sid = lax.axis_index("s")
        seg = cid * 8 + sid

        @pl.when(sid < 8)
        def _active():
            pltpu.sync_copy(starts_hbm, sv)
            pltpu.sync_copy(ends_hbm, ev)
            sv_vec = sv[...]
            ev_vec = ev[...]
            for K in range(SEGS):
                @pl.when(seg == K)
                def _pick(K=K):
                    se_smem[0] = sv_vec[K]
                    se_smem[1] = ev_vec[K]
            start = se_smem[0]
            end = se_smem[1]
            base = jnp.minimum((start // 8) * 8, MAX_BASE)

            pltpu.sync_copy(scores_hbm.at[pl.ds(base, WIN)], win_v)

            lane = lax.broadcasted_iota(jnp.int32, (LANES,), 0)
            s_acc = jnp.zeros((LANES,), jnp.float32)
            m_acc = jnp.full((LANES,), -jnp.inf, jnp.float32)

            for i in range(WIN // LANES):
                x = win_v[pl.ds(i * LANES, LANES)]
                g = (base + i * LANES) + lane
                inseg = (g >= start) & (g < end)
                s_acc = s_acc + jnp.where(inseg, x, 0.0)
                m_acc = jnp.maximum(m_acc, jnp.where(inseg, x, -jnp.inf))

            s_ref[...] = s_acc
            m_ref[...] = m_acc

            def allreduce(v, op):
                dn = lax.GatherDimensionNumbers(
                    offset_dims=(), collapsed_slice_dims=(0,),
                    start_index_map=(0,))
                for step in (1, 2, 4, 8):
                    idx = jnp.bitwise_xor(lane, step).reshape(LANES, 1)
                    perm = lax.gather(
                        v, idx, dn, (1,),
                        mode=lax.GatherScatterMode.PROMISE_IN_BOUNDS)
                    v = op(v, perm)
                return v

            sum_vec = allreduce(s_ref[...], jnp.add)
            max_vec = allreduce(m_ref[...], jnp.maximum)
            count = (end - start).astype(jnp.float32)
            res_v[...] = (WEIGHT * max_vec
                          + (1.0 - WEIGHT) * (sum_vec / count))
            pltpu.sync_copy(res_v, out_hbm.at[seg])

    out = k(scores, starts, ends)
    return out[:, 15]


def kernel(nodes, goal, num_nodes, W1, b1, W2, b2):
    nn = num_nodes.astype(jnp.int32)
    ends = jnp.cumsum(nn)
    starts = ends - nn
    scores = _tc_scores(nodes, goal, W1, b1, W2, b2)
    return _sc_pool(scores, starts, ends)


# speedup vs baseline: 2.4610x; 2.4610x over previous
import jax
import jax.numpy as jnp
from jax import lax
from jax.experimental import pallas as pl
from jax.experimental.pallas import tpu as pltpu

TOKENS = 32768
SEGS = 16
NODE_D = 512
GOAL_D = 128
CH = 1024
NBUF = 8
NCH = TOKENS // CH
OUTER = NCH // NBUF
WEIGHT = 0.7


def _body(starts_ref, ends_ref, w1n_ref, w1g_ref, b1_ref, w2_ref, b2_ref,
          counts_ref, nodes_hbm, goal_hbm, out_ref,
          nbuf_ref, gbuf_ref, sum_ref, max_ref, sems):

    H = CH // 2

    def node_copy_lo(c, b):
        return pltpu.make_async_copy(
            nodes_hbm.at[pl.ds(c * CH, H), :],
            nbuf_ref.at[b, pl.ds(0, H), :], sems.at[b, 0])

    def node_copy_hi(c, b):
        return pltpu.make_async_copy(
            nodes_hbm.at[pl.ds(c * CH + H, H), :],
            nbuf_ref.at[b, pl.ds(H, H), :], sems.at[b, 2])

    def goal_copy(c, b):
        return pltpu.make_async_copy(
            goal_hbm.at[pl.ds(c * CH, CH), :], gbuf_ref.at[b], sems.at[b, 1])

    sum_ref[...] = jnp.zeros_like(sum_ref)
    max_ref[...] = jnp.full_like(max_ref, -jnp.inf)

    for b in range(NBUF):
        node_copy_lo(b, b).start()
        node_copy_hi(b, b).start()
        goal_copy(b, b).start()

    def outer(i, _):
        for b in range(NBUF):
            c = i * NBUF + b
            node_copy_lo(c, b).wait()
            node_copy_hi(c, b).wait()
            goal_copy(c, b).wait()

            h = jnp.dot(nbuf_ref[b], w1n_ref[...],
                        preferred_element_type=jnp.float32)
            h = h + jnp.dot(gbuf_ref[b], w1g_ref[...],
                            preferred_element_type=jnp.float32)
            h = jnp.maximum(h + b1_ref[...], 0.0)
            scores = jnp.sum(h * w2_ref[...], axis=1, keepdims=True)

            gidx = (jax.lax.broadcasted_iota(jnp.int32, (CH, SEGS), 0)
                    + c * CH)
            mask = (gidx >= starts_ref[...]) & (gidx < ends_ref[...])
            sum_ref[...] += jnp.sum(jnp.where(mask, scores, 0.0),
                                    axis=0, keepdims=True)
            max_ref[...] = jnp.maximum(
                max_ref[...],
                jnp.max(jnp.where(mask, scores, -jnp.inf),
                        axis=0, keepdims=True))

            @pl.when(i < OUTER - 1)
            def _prefetch():
                node_copy_lo(c + NBUF, b).start()
                node_copy_hi(c + NBUF, b).start()
                goal_copy(c + NBUF, b).start()
        return 0

    lax.fori_loop(0, OUTER, outer, 0)

    mean = sum_ref[...] / counts_ref[...]
    out_ref[...] = WEIGHT * max_ref[...] + (1.0 - WEIGHT) * mean + b2_ref[...]


def kernel(nodes, goal, num_nodes, W1, b1, W2, b2):
    nn = num_nodes.astype(jnp.int32)
    ends = jnp.cumsum(nn)
    starts = ends - nn
    counts = nn.astype(jnp.float32).reshape(1, SEGS)

    w1nT = W1[:, :NODE_D].T
    w1gT = W1[:, NODE_D:].T

    full = lambda i: (0, 0)
    out = pl.pallas_call(
        _body,
        grid=(1,),
        in_specs=[
            pl.BlockSpec((1, SEGS), full),
            pl.BlockSpec((1, SEGS), full),
            pl.BlockSpec((NODE_D, SEGS), full),
            pl.BlockSpec((GOAL_D, SEGS), full),
            pl.BlockSpec((1, SEGS), full),
            pl.BlockSpec((1, SEGS), full),
            pl.BlockSpec((1, 1), full),
            pl.BlockSpec((1, SEGS), full),
            pl.BlockSpec(memory_space=pl.ANY),
            pl.BlockSpec(memory_space=pl.ANY),
        ],
        out_specs=pl.BlockSpec((1, SEGS), full),
        out_shape=jax.ShapeDtypeStruct((1, SEGS), jnp.float32),
        scratch_shapes=[
            pltpu.VMEM((NBUF, CH, NODE_D), jnp.float32),
            pltpu.VMEM((NBUF, CH, GOAL_D), jnp.float32),
            pltpu.VMEM((1, SEGS), jnp.float32),
            pltpu.VMEM((1, SEGS), jnp.float32),
            pltpu.SemaphoreType.DMA((NBUF, 3)),
        ],
        compiler_params=pltpu.CompilerParams(
            dimension_semantics=("arbitrary",)),
    )(starts.reshape(1, SEGS), ends.reshape(1, SEGS), w1nT, w1gT,
      b1.reshape(1, SEGS), W2.reshape(1, SEGS), b2.reshape(1, 1), counts,
      nodes, goal)
    return out.reshape(SEGS)
